# Initial kernel scaffold; baseline (speedup 1.0000x reference)
import jax
import jax.numpy as jnp
from jax.experimental import pallas as pl


def kernel(x, edge_index, batch, W, att_src, att_dst, bias, proj_W, proj_b):
    raise NotImplementedError("write your pallas kernel here")


# speedup vs baseline: 13.7627x; 13.7627x over previous
import functools

import jax
import jax.numpy as jnp
from jax import lax
from jax.experimental import pallas as pl
from jax.experimental.pallas import tpu as pltpu
from jax.experimental.pallas import tpu_sc as plsc

N = 10000
E = 320000
D = 128
DH = D // 2
G = 64
ACCW = 80
CH = 128
NCH = E // CH
NC = 2
NS = 16
NP = 10240
NR = NP // NS


_NB1 = 1000


def _node_proj_body(x_ref, w_ref, asv_ref, adv_ref,
                    lo_ref, hi_ref, s_ref, d_ref):
    xh = jnp.dot(x_ref[...], w_ref[...], preferred_element_type=jnp.float32)
    lo_ref[...] = xh[:, :DH]
    hi_ref[...] = xh[:, DH:]
    s_ref[...] = jnp.sum(xh * asv_ref[...], axis=1, keepdims=True)
    d_ref[...] = jnp.sum(xh * adv_ref[...], axis=1, keepdims=True)


def _node_proj(x, W, att_src_row, att_dst_row):
    return pl.pallas_call(
        _node_proj_body,
        grid=(N // _NB1,),
        in_specs=[
            pl.BlockSpec((_NB1, D), lambda i: (i, 0)),
            pl.BlockSpec((D, D), lambda i: (0, 0)),
            pl.BlockSpec((1, D), lambda i: (0, 0)),
            pl.BlockSpec((1, D), lambda i: (0, 0)),
        ],
        out_specs=[
            pl.BlockSpec((_NB1, DH), lambda i: (i, 0)),
            pl.BlockSpec((_NB1, DH), lambda i: (i, 0)),
            pl.BlockSpec((_NB1, 1), lambda i: (i, 0)),
            pl.BlockSpec((_NB1, 1), lambda i: (i, 0)),
        ],
        out_shape=[
            jax.ShapeDtypeStruct((N, DH), jnp.float32),
            jax.ShapeDtypeStruct((N, DH), jnp.float32),
            jax.ShapeDtypeStruct((N, 1), jnp.float32),
            jax.ShapeDtypeStruct((N, 1), jnp.float32),
        ],
    )(x, W, att_src_row, att_dst_row)


def _edge_body(edge_ref, asrc_hbm, adst_hbm, lo_hbm, hi_hbm, out_ref,
               asrc_v, adst_v, src_b, dst_b, ex_b, rows_b, scaled_b,
               acc_sh, sem):
    c = lax.axis_index("c")
    s = lax.axis_index("s")

    pltpu.sync_copy(asrc_hbm, asrc_v)
    pltpu.sync_copy(adst_hbm, adst_v)

    def _zb(j, cy):
        for q in range(ACCW // 16):
            scaled_b[j, pl.ds(16 * q, 16)] = jnp.zeros((16,), jnp.float32)
        return cy
    lax.fori_loop(0, CH, _zb, 0)
    r0 = s * NR
    for t in range(NR // CH):
        pltpu.sync_copy(scaled_b, acc_sh.at[pl.ds(r0 + CH * t, CH)])
    plsc.subcore_barrier()

    nch = jnp.where(s < NCH % NS, NCH // NS + 1, NCH // NS)

    def _chunk(i, cy):
        off = (s + NS * i) * CH
        pltpu.sync_copy(edge_ref.at[0, pl.ds(off, CH)], src_b)
        pltpu.sync_copy(edge_ref.at[1, pl.ds(off, CH)], dst_b)

        @pl.when(c == 0)
        def _():
            pltpu.async_copy(lo_hbm.at[src_b], rows_b, sem).wait()

        @pl.when(c == 1)
        def _():
            pltpu.async_copy(hi_hbm.at[src_b], rows_b, sem).wait()

        for v in range(CH // 16):
            sv = src_b[pl.ds(16 * v, 16)]
            dv = dst_b[pl.ds(16 * v, 16)]
            al = (plsc.load_gather(asrc_v, [sv])
                  + plsc.load_gather(adst_v, [dv]))
            al = jnp.where(al >= 0.0, al, 0.2 * al)
            ex_b[pl.ds(16 * v, 16)] = jnp.exp(al)

        def _scale(j, cy2):
            e = plsc.load_gather(ex_b, [jnp.full((16,), j, jnp.int32)])
            for q in range(DH // 16):
                scaled_b[j, pl.ds(16 * q, 16)] = (
                    e * rows_b[j, pl.ds(16 * q, 16)])
            lane = lax.iota(jnp.int32, 16)
            scaled_b[j, pl.ds(DH, 16)] = jnp.where(lane == 0, e, 0.0)
            return cy2
        lax.fori_loop(0, CH, _scale, 0)
        pltpu.sync_copy(scaled_b, acc_sh.at[dst_b], add=True)
        return cy
    lax.fori_loop(0, nch, _chunk, 0)

    plsc.subcore_barrier()
    for t in range(NR // CH):
        pltpu.sync_copy(acc_sh.at[pl.ds(r0 + CH * t, CH)], scaled_b)
        pltpu.sync_copy(scaled_b, out_ref.at[c, pl.ds(r0 + CH * t, CH)])


@functools.cache
def _edge_pass():
    return pl.kernel(
        _edge_body,
        mesh=plsc.VectorSubcoreMesh(core_axis_name="c", subcore_axis_name="s",
                                    num_cores=NC, num_subcores=NS),
        out_type=jax.ShapeDtypeStruct((NC, NP, ACCW), jnp.float32),
        scratch_types=[
            pltpu.VMEM((N,), jnp.float32),
            pltpu.VMEM((N,), jnp.float32),
            pltpu.VMEM((CH,), jnp.int32),
            pltpu.VMEM((CH,), jnp.int32),
            pltpu.VMEM((CH,), jnp.float32),
            pltpu.VMEM((CH, DH), jnp.float32),
            pltpu.VMEM((CH, ACCW), jnp.float32),
            pltpu.VMEM_SHARED((NP, ACCW), jnp.float32),
            pltpu.SemaphoreType.DMA,
        ],
        compiler_params=pltpu.CompilerParams(needs_layout_passes=False,
                                             use_tc_tiling_on_sc=False),
    )


_NB3 = 400
_NG3 = N // _NB3


def _final_body(acc0_ref, acc1_ref, lo_ref, hi_ref, as_ref, ad_ref, b_ref,
                bias_ref, pw_ref, pb_ref, pooled_ref, s_ref,
                sums_ref, cnt_ref):
    i = pl.program_id(0)
    al = as_ref[...] + ad_ref[...]
    ex_self = jnp.exp(jnp.where(al >= 0.0, al, 0.2 * al))
    xh = jnp.concatenate([lo_ref[...], hi_ref[...]], axis=1)
    num = jnp.concatenate([acc0_ref[:, :DH], acc1_ref[:, :DH]], axis=1)
    num = num + ex_self * xh
    den = acc0_ref[:, DH:DH + 1] + ex_self + 1e-16
    xho = num / den + bias_ref[...]
    logits = jnp.sum(xho * pw_ref[...], axis=1, keepdims=True) + pb_ref[...]
    sc = 1.0 / (1.0 + jnp.exp(-logits))
    s_ref[...] = sc
    xw = sc * xho
    onehot = (b_ref[...] == lax.broadcasted_iota(jnp.int32, (_NB3, G), 1)
              ).astype(jnp.float32)
    psum = lax.dot_general(onehot, xw, (((0,), (0,)), ((), ())),
                           preferred_element_type=jnp.float32)
    pcnt = lax.dot_general(onehot, jnp.ones((_NB3, 1), jnp.float32),
                           (((0,), (0,)), ((), ())),
                           preferred_element_type=jnp.float32)

    @pl.when(i == 0)
    def _():
        sums_ref[...] = psum
        cnt_ref[...] = pcnt

    @pl.when(i > 0)
    def _():
        sums_ref[...] += psum
        cnt_ref[...] += pcnt

    @pl.when(i == _NG3 - 1)
    def _():
        pooled_ref[...] = sums_ref[...] / jnp.maximum(cnt_ref[...], 1.0)


def _finalize(acc0, acc1, xh_lo, xh_hi, a_src, a_dst, batch2d,
              bias_row, pw_row, pb):
    return pl.pallas_call(
        _final_body,
        grid=(_NG3,),
        in_specs=[
            pl.BlockSpec((_NB3, ACCW), lambda i: (i, 0)),
            pl.BlockSpec((_NB3, ACCW), lambda i: (i, 0)),
            pl.BlockSpec((_NB3, DH), lambda i: (i, 0)),
            pl.BlockSpec((_NB3, DH), lambda i: (i, 0)),
            pl.BlockSpec((_NB3, 1), lambda i: (i, 0)),
            pl.BlockSpec((_NB3, 1), lambda i: (i, 0)),
            pl.BlockSpec((_NB3, 1), lambda i: (i, 0)),
            pl.BlockSpec((1, D), lambda i: (0, 0)),
            pl.BlockSpec((1, D), lambda i: (0, 0)),
            pl.BlockSpec((1, 1), lambda i: (0, 0)),
        ],
        out_specs=[
            pl.BlockSpec((G, D), lambda i: (0, 0)),
            pl.BlockSpec((_NB3, 1), lambda i: (i, 0)),
        ],
        out_shape=[
            jax.ShapeDtypeStruct((G, D), jnp.float32),
            jax.ShapeDtypeStruct((N, 1), jnp.float32),
        ],
        scratch_shapes=[
            pltpu.VMEM((G, D), jnp.float32),
            pltpu.VMEM((G, 1), jnp.float32),
        ],
    )(acc0, acc1, xh_lo, xh_hi, a_src, a_dst, batch2d, bias_row, pw_row, pb)


@jax.jit
def kernel(x, edge_index, batch, W, att_src, att_dst, bias, proj_W, proj_b):
    xh_lo, xh_hi, a_src, a_dst = _node_proj(
        x, W, att_src.reshape(1, D), att_dst.reshape(1, D))
    acc = _edge_pass()(edge_index, a_src.reshape(N), a_dst.reshape(N),
                       xh_lo, xh_hi)
    acc = acc[:, :N]
    pooled, scores = _finalize(
        acc[0], acc[1], xh_lo, xh_hi, a_src, a_dst, batch.reshape(N, 1),
        bias.reshape(1, D), proj_W.reshape(1, D), proj_b.reshape(1, 1))
    return (pooled, scores)


# speedup vs baseline: 19.0264x; 1.3825x over previous
import functools

import jax
import jax.numpy as jnp
from jax import lax
from jax.experimental import pallas as pl
from jax.experimental.pallas import tpu as pltpu
from jax.experimental.pallas import tpu_sc as plsc

N = 10000
E = 320000
D = 128
DH = D // 2
G = 64
ACCW = 80
CH = 128
NCH = E // CH
NC = 2
NS = 16
NCHW = (NCH // NS) & ~1
NREM = NCH - NCHW * NS
NP = 10240
NR = NP // NS


_NB1 = 1000


def _node_proj_body(x_ref, w_ref, asv_ref, adv_ref, xh2_ref, s_ref, d_ref):
    xh = jnp.dot(x_ref[...], w_ref[...], preferred_element_type=jnp.float32)
    xh2_ref[0] = xh[:, :DH]
    xh2_ref[1] = xh[:, DH:]
    s_ref[...] = jnp.sum(xh * asv_ref[...], axis=1, keepdims=True)
    d_ref[...] = jnp.sum(xh * adv_ref[...], axis=1, keepdims=True)


def _node_proj(x, W, att_src_row, att_dst_row):
    return pl.pallas_call(
        _node_proj_body,
        grid=(N // _NB1,),
        in_specs=[
            pl.BlockSpec((_NB1, D), lambda i: (i, 0)),
            pl.BlockSpec((D, D), lambda i: (0, 0)),
            pl.BlockSpec((1, D), lambda i: (0, 0)),
            pl.BlockSpec((1, D), lambda i: (0, 0)),
        ],
        out_specs=[
            pl.BlockSpec((2, _NB1, DH), lambda i: (0, i, 0)),
            pl.BlockSpec((_NB1, 1), lambda i: (i, 0)),
            pl.BlockSpec((_NB1, 1), lambda i: (i, 0)),
        ],
        out_shape=[
            jax.ShapeDtypeStruct((2, N, DH), jnp.float32),
            jax.ShapeDtypeStruct((N, 1), jnp.float32),
            jax.ShapeDtypeStruct((N, 1), jnp.float32),
        ],
    )(x, W, att_src_row, att_dst_row)


def _edge_body(edge_ref, asrc_hbm, adst_hbm, xh2_hbm, out_ref,
               asrc_v, adst_v,
               src_b0, dst_b0, rows_b0, scaled_b0, dsc_b0,
               src_b1, dst_b1, rows_b1, scaled_b1, dsc_b1,
               ex_b, acc_sh, se0, se1, sg, ss0, ss1):
    c = lax.axis_index("c")
    s = lax.axis_index("s")

    pltpu.sync_copy(asrc_hbm, asrc_v)
    pltpu.sync_copy(adst_hbm, adst_v)

    def _zb(j, cy):
        for q in range(ACCW // 16):
            scaled_b0[j, pl.ds(16 * q, 16)] = jnp.zeros((16,), jnp.float32)
        return cy
    lax.fori_loop(0, CH, _zb, 0)
    r0 = s * NR
    for t in range(NR // CH):
        pltpu.sync_copy(scaled_b0, acc_sh.at[pl.ds(r0 + CH * t, CH)])
    plsc.subcore_barrier()

    bufs = ((src_b0, dst_b0, rows_b0, scaled_b0, dsc_b0, se0, ss0),
            (src_b1, dst_b1, rows_b1, scaled_b1, dsc_b1, se1, ss1))

    def _start_edges(i, sbuf, dbuf, se):
        off = (s + NS * i) * CH
        pltpu.async_copy(edge_ref.at[0, pl.ds(off, CH)], sbuf, se)
        pltpu.async_copy(edge_ref.at[1, pl.ds(off, CH)], dbuf, se)

    def _wait_edges(sbuf, dbuf, se):
        pltpu.make_async_copy(edge_ref.at[0, pl.ds(0, CH)], sbuf, se).wait()
        pltpu.make_async_copy(edge_ref.at[1, pl.ds(0, CH)], dbuf, se).wait()

    def _compute_ex(sbuf, dbuf):
        for v in range(CH // 16):
            sv = sbuf[pl.ds(16 * v, 16)]
            dv = dbuf[pl.ds(16 * v, 16)]
            al = (plsc.load_gather(asrc_v, [sv])
                  + plsc.load_gather(adst_v, [dv]))
            al = jnp.where(al >= 0.0, al, 0.2 * al)
            ex_b[pl.ds(16 * v, 16)] = jnp.exp(al)

    def _scale(rbuf, cbuf):
        def _row(j, cy):
            e = plsc.load_gather(ex_b, [jnp.full((16,), j, jnp.int32)])
            for q in range(DH // 16):
                cbuf[j, pl.ds(16 * q, 16)] = e * rbuf[j, pl.ds(16 * q, 16)]
            lane = lax.iota(jnp.int32, 16)
            cbuf[j, pl.ds(DH, 16)] = jnp.where(lane == 0, e, 0.0)
            return cy
        lax.fori_loop(0, CH, _row, 0, unroll=4)

    _start_edges(0, src_b0, dst_b0, se0)
    _start_edges(1, src_b1, dst_b1, se1)

    def _pair(k, cy):
        for b in range(2):
            i = 2 * k + b
            sbuf, dbuf, rbuf, cbuf, dsc, se, ss = bufs[b]
            _wait_edges(sbuf, dbuf, se)
            gat = pltpu.async_copy(xh2_hbm.at[c].at[sbuf], rbuf, sg)
            _compute_ex(sbuf, dbuf)

            @pl.when(k >= 1)
            def _():
                pltpu.make_async_copy(cbuf, acc_sh.at[dsc], ss).wait()
            for v in range(CH // 16):
                dsc[pl.ds(16 * v, 16)] = dbuf[pl.ds(16 * v, 16)]
            gat.wait()

            @pl.when(k < NCHW // 2 - 1)
            def _():
                _start_edges(i + 2, sbuf, dbuf, se)
            _scale(rbuf, cbuf)
            pltpu.async_copy(cbuf, acc_sh.at[dsc], ss, add=True)
        return cy
    lax.fori_loop(0, NCHW // 2, _pair, 0)

    pltpu.make_async_copy(scaled_b0, acc_sh.at[dsc_b0], ss0).wait()
    pltpu.make_async_copy(scaled_b1, acc_sh.at[dsc_b1], ss1).wait()

    @pl.when(s < NREM)
    def _():
        off = (NCHW * NS + s) * CH
        pltpu.sync_copy(edge_ref.at[0, pl.ds(off, CH)], src_b0)
        pltpu.sync_copy(edge_ref.at[1, pl.ds(off, CH)], dst_b0)
        pltpu.async_copy(xh2_hbm.at[c].at[src_b0], rows_b0, sg).wait()
        _compute_ex(src_b0, dst_b0)
        _scale(rows_b0, scaled_b0)
        pltpu.sync_copy(scaled_b0, acc_sh.at[dst_b0], add=True)

    plsc.subcore_barrier()
    for t in range(NR // CH):
        pltpu.sync_copy(acc_sh.at[pl.ds(r0 + CH * t, CH)], scaled_b0)
        pltpu.sync_copy(scaled_b0, out_ref.at[c, pl.ds(r0 + CH * t, CH)])


@functools.cache
def _edge_pass():
    return pl.kernel(
        _edge_body,
        mesh=plsc.VectorSubcoreMesh(core_axis_name="c", subcore_axis_name="s",
                                    num_cores=NC, num_subcores=NS),
        out_type=jax.ShapeDtypeStruct((NC, NP, ACCW), jnp.float32),
        scratch_types=[
            pltpu.VMEM((N,), jnp.float32),
            pltpu.VMEM((N,), jnp.float32),
            pltpu.VMEM((CH,), jnp.int32),
            pltpu.VMEM((CH,), jnp.int32),
            pltpu.VMEM((CH, DH), jnp.float32),
            pltpu.VMEM((CH, ACCW), jnp.float32),
            pltpu.VMEM((CH,), jnp.int32),
            pltpu.VMEM((CH,), jnp.int32),
            pltpu.VMEM((CH,), jnp.int32),
            pltpu.VMEM((CH, DH), jnp.float32),
            pltpu.VMEM((CH, ACCW), jnp.float32),
            pltpu.VMEM((CH,), jnp.int32),
            pltpu.VMEM((CH,), jnp.float32),
            pltpu.VMEM_SHARED((NP, ACCW), jnp.float32),
            pltpu.SemaphoreType.DMA,
            pltpu.SemaphoreType.DMA,
            pltpu.SemaphoreType.DMA,
            pltpu.SemaphoreType.DMA,
            pltpu.SemaphoreType.DMA,
        ],
        compiler_params=pltpu.CompilerParams(needs_layout_passes=False,
                                             use_tc_tiling_on_sc=False),
    )


_NB3 = 400
_NG3 = N // _NB3


def _final_body(acc0_ref, acc1_ref, lo_ref, hi_ref, as_ref, ad_ref, b_ref,
                bias_ref, pw_ref, pb_ref, pooled_ref, s_ref,
                sums_ref, cnt_ref):
    i = pl.program_id(0)
    al = as_ref[...] + ad_ref[...]
    ex_self = jnp.exp(jnp.where(al >= 0.0, al, 0.2 * al))
    xh = jnp.concatenate([lo_ref[0], hi_ref[0]], axis=1)
    num = jnp.concatenate([acc0_ref[0, :, :DH], acc1_ref[0, :, :DH]], axis=1)
    num = num + ex_self * xh
    den = acc0_ref[0, :, DH:DH + 1] + ex_self + 1e-16
    xho = num / den + bias_ref[...]
    logits = jnp.sum(xho * pw_ref[...], axis=1, keepdims=True) + pb_ref[...]
    sc = 1.0 / (1.0 + jnp.exp(-logits))
    s_ref[...] = sc
    xw = sc * xho
    onehot = (b_ref[...] == lax.broadcasted_iota(jnp.int32, (_NB3, G), 1)
              ).astype(jnp.float32)
    psum = lax.dot_general(onehot, xw, (((0,), (0,)), ((), ())),
                           preferred_element_type=jnp.float32)
    pcnt = lax.dot_general(onehot, jnp.ones((_NB3, 1), jnp.float32),
                           (((0,), (0,)), ((), ())),
                           preferred_element_type=jnp.float32)

    @pl.when(i == 0)
    def _():
        sums_ref[...] = psum
        cnt_ref[...] = pcnt

    @pl.when(i > 0)
    def _():
        sums_ref[...] += psum
        cnt_ref[...] += pcnt

    @pl.when(i == _NG3 - 1)
    def _():
        pooled_ref[...] = sums_ref[...] / jnp.maximum(cnt_ref[...], 1.0)


def _finalize(acc, xh2, a_src, a_dst, batch2d, bias_row, pw_row, pb):
    return pl.pallas_call(
        _final_body,
        grid=(_NG3,),
        in_specs=[
            pl.BlockSpec((1, _NB3, ACCW), lambda i: (0, i, 0)),
            pl.BlockSpec((1, _NB3, ACCW), lambda i: (1, i, 0)),
            pl.BlockSpec((1, _NB3, DH), lambda i: (0, i, 0)),
            pl.BlockSpec((1, _NB3, DH), lambda i: (1, i, 0)),
            pl.BlockSpec((_NB3, 1), lambda i: (i, 0)),
            pl.BlockSpec((_NB3, 1), lambda i: (i, 0)),
            pl.BlockSpec((_NB3, 1), lambda i: (i, 0)),
            pl.BlockSpec((1, D), lambda i: (0, 0)),
            pl.BlockSpec((1, D), lambda i: (0, 0)),
            pl.BlockSpec((1, 1), lambda i: (0, 0)),
        ],
        out_specs=[
            pl.BlockSpec((G, D), lambda i: (0, 0)),
            pl.BlockSpec((_NB3, 1), lambda i: (i, 0)),
        ],
        out_shape=[
            jax.ShapeDtypeStruct((G, D), jnp.float32),
            jax.ShapeDtypeStruct((N, 1), jnp.float32),
        ],
        scratch_shapes=[
            pltpu.VMEM((G, D), jnp.float32),
            pltpu.VMEM((G, 1), jnp.float32),
        ],
    )(acc, acc, xh2, xh2, a_src, a_dst, batch2d, bias_row, pw_row, pb)


@jax.jit
def kernel(x, edge_index, batch, W, att_src, att_dst, bias, proj_W, proj_b):
    xh2, a_src, a_dst = _node_proj(
        x, W, att_src.reshape(1, D), att_dst.reshape(1, D))
    acc = _edge_pass()(edge_index, a_src.reshape(N), a_dst.reshape(N), xh2)
    pooled, scores = _finalize(
        acc, xh2, a_src, a_dst, batch.reshape(N, 1),
        bias.reshape(1, D), proj_W.reshape(1, D), proj_b.reshape(1, 1))
    return (pooled, scores)


# speedup vs baseline: 23.5649x; 1.2385x over previous
import functools

import jax
import jax.numpy as jnp
from jax import lax
from jax.experimental import pallas as pl
from jax.experimental.pallas import tpu as pltpu
from jax.experimental.pallas import tpu_sc as plsc

N = 10000
E = 320000
D = 128
DH = D // 2
G = 64
ACCW = 80
CH = 128
NCH = E // CH
NC = 2
NS = 16
NCHW = (NCH // NS) & ~1
NREM = NCH - NCHW * NS
NP = 10240
NR = NP // NS


_NB1 = 1000


def _node_proj_body(x_ref, w_ref, asv_ref, adv_ref, xh2_ref, s_ref, d_ref):
    xh = jnp.dot(x_ref[...], w_ref[...], preferred_element_type=jnp.float32)
    xh2_ref[0] = xh[:, :DH]
    xh2_ref[1] = xh[:, DH:]
    s_ref[...] = jnp.sum(xh * asv_ref[...], axis=1, keepdims=True)
    d_ref[...] = jnp.sum(xh * adv_ref[...], axis=1, keepdims=True)


def _node_proj(x, W, att_src_row, att_dst_row):
    return pl.pallas_call(
        _node_proj_body,
        grid=(N // _NB1,),
        in_specs=[
            pl.BlockSpec((_NB1, D), lambda i: (i, 0)),
            pl.BlockSpec((D, D), lambda i: (0, 0)),
            pl.BlockSpec((1, D), lambda i: (0, 0)),
            pl.BlockSpec((1, D), lambda i: (0, 0)),
        ],
        out_specs=[
            pl.BlockSpec((2, _NB1, DH), lambda i: (0, i, 0)),
            pl.BlockSpec((_NB1, 1), lambda i: (i, 0)),
            pl.BlockSpec((_NB1, 1), lambda i: (i, 0)),
        ],
        out_shape=[
            jax.ShapeDtypeStruct((2, N, DH), jnp.float32),
            jax.ShapeDtypeStruct((N, 1), jnp.float32),
            jax.ShapeDtypeStruct((N, 1), jnp.float32),
        ],
    )(x, W, att_src_row, att_dst_row)


def _edge_body(edge_ref, asrc_hbm, adst_hbm, xh2_hbm, out_ref,
               asrc_v, adst_v,
               src_b0, dst_b0, rows_b0, scaled_b0, dsc_b0,
               src_b1, dst_b1, rows_b1, scaled_b1, dsc_b1,
               ex_b, acc_sh, se0, se1, sg, ss0, ss1):
    c = lax.axis_index("c")
    s = lax.axis_index("s")

    pltpu.sync_copy(asrc_hbm, asrc_v)
    pltpu.sync_copy(adst_hbm, adst_v)

    def _zb(j, cy):
        for q in range(ACCW // 16):
            scaled_b0[j, pl.ds(16 * q, 16)] = jnp.zeros((16,), jnp.float32)
        return cy
    lax.fori_loop(0, CH, _zb, 0)
    r0 = s * NR
    for t in range(NR // CH):
        pltpu.sync_copy(scaled_b0, acc_sh.at[pl.ds(r0 + CH * t, CH)])
    plsc.subcore_barrier()

    bufs = ((src_b0, dst_b0, rows_b0, scaled_b0, dsc_b0, se0, ss0),
            (src_b1, dst_b1, rows_b1, scaled_b1, dsc_b1, se1, ss1))

    def _start_edges(i, sbuf, dbuf, se):
        off = (s + NS * i) * CH
        pltpu.async_copy(edge_ref.at[0, pl.ds(off, CH)], sbuf, se)
        pltpu.async_copy(edge_ref.at[1, pl.ds(off, CH)], dbuf, se)

    def _wait_edges(sbuf, dbuf, se):
        pltpu.make_async_copy(edge_ref.at[0, pl.ds(0, CH)], sbuf, se).wait()
        pltpu.make_async_copy(edge_ref.at[1, pl.ds(0, CH)], dbuf, se).wait()

    def _compute_ex(sbuf, dbuf):
        for v in range(CH // 16):
            sv = sbuf[pl.ds(16 * v, 16)]
            dv = dbuf[pl.ds(16 * v, 16)]
            al = (plsc.load_gather(asrc_v, [sv])
                  + plsc.load_gather(adst_v, [dv]))
            al = jnp.where(al >= 0.0, al, 0.2 * al)
            ex_b[pl.ds(16 * v, 16)] = jnp.exp(al)

    def _scale(rbuf, cbuf):
        def _row(j, cy):
            e = plsc.load_gather(ex_b, [jnp.full((16,), j, jnp.int32)])
            for q in range(DH // 16):
                cbuf[j, pl.ds(16 * q, 16)] = e * rbuf[j, pl.ds(16 * q, 16)]
            lane = lax.iota(jnp.int32, 16)
            cbuf[j, pl.ds(DH, 16)] = jnp.where(lane == 0, e, 0.0)
            return cy
        lax.fori_loop(0, CH, _row, 0, unroll=4)

    _start_edges(0, src_b0, dst_b0, se0)
    _start_edges(1, src_b1, dst_b1, se1)
    _wait_edges(src_b0, dst_b0, se0)
    pltpu.async_copy(xh2_hbm.at[c].at[src_b0], rows_b0, sg)

    def _pair(k, cy):
        for b in range(2):
            i = 2 * k + b
            sbuf, dbuf, rbuf, cbuf, dsc, se, ss = bufs[b]
            nsbuf, ndbuf, nrbuf, _, _, nse, _ = bufs[1 - b]

            @pl.when(k >= 1)
            def _():
                pltpu.make_async_copy(cbuf, acc_sh.at[dsc], ss).wait()
            pltpu.make_async_copy(xh2_hbm.at[c].at[sbuf], rbuf, sg).wait()
            _compute_ex(sbuf, dbuf)
            for v in range(CH // 16):
                dsc[pl.ds(16 * v, 16)] = dbuf[pl.ds(16 * v, 16)]

            @pl.when(k < NCHW // 2 - 1)
            def _():
                _start_edges(i + 2, sbuf, dbuf, se)

            def _next_gather():
                _wait_edges(nsbuf, ndbuf, nse)
                pltpu.async_copy(xh2_hbm.at[c].at[nsbuf], nrbuf, sg)
            if b == 0:
                _next_gather()
            else:
                pl.when(k < NCHW // 2 - 1)(_next_gather)
            _scale(rbuf, cbuf)
            pltpu.async_copy(cbuf, acc_sh.at[dsc], ss, add=True)
        return cy
    lax.fori_loop(0, NCHW // 2, _pair, 0)

    pltpu.make_async_copy(scaled_b0, acc_sh.at[dsc_b0], ss0).wait()
    pltpu.make_async_copy(scaled_b1, acc_sh.at[dsc_b1], ss1).wait()

    @pl.when(s < NREM)
    def _():
        off = (NCHW * NS + s) * CH
        pltpu.sync_copy(edge_ref.at[0, pl.ds(off, CH)], src_b0)
        pltpu.sync_copy(edge_ref.at[1, pl.ds(off, CH)], dst_b0)
        pltpu.async_copy(xh2_hbm.at[c].at[src_b0], rows_b0, sg).wait()
        _compute_ex(src_b0, dst_b0)
        _scale(rows_b0, scaled_b0)
        pltpu.sync_copy(scaled_b0, acc_sh.at[dst_b0], add=True)

    plsc.subcore_barrier()
    for t in range(NR // CH):
        pltpu.sync_copy(acc_sh.at[pl.ds(r0 + CH * t, CH)], scaled_b0)
        pltpu.sync_copy(scaled_b0, out_ref.at[c, pl.ds(r0 + CH * t, CH)])


@functools.cache
def _edge_pass():
    return pl.kernel(
        _edge_body,
        mesh=plsc.VectorSubcoreMesh(core_axis_name="c", subcore_axis_name="s",
                                    num_cores=NC, num_subcores=NS),
        out_type=jax.ShapeDtypeStruct((NC, NP, ACCW), jnp.float32),
        scratch_types=[
            pltpu.VMEM((N,), jnp.float32),
            pltpu.VMEM((N,), jnp.float32),
            pltpu.VMEM((CH,), jnp.int32),
            pltpu.VMEM((CH,), jnp.int32),
            pltpu.VMEM((CH, DH), jnp.float32),
            pltpu.VMEM((CH, ACCW), jnp.float32),
            pltpu.VMEM((CH,), jnp.int32),
            pltpu.VMEM((CH,), jnp.int32),
            pltpu.VMEM((CH,), jnp.int32),
            pltpu.VMEM((CH, DH), jnp.float32),
            pltpu.VMEM((CH, ACCW), jnp.float32),
            pltpu.VMEM((CH,), jnp.int32),
            pltpu.VMEM((CH,), jnp.float32),
            pltpu.VMEM_SHARED((NP, ACCW), jnp.float32),
            pltpu.SemaphoreType.DMA,
            pltpu.SemaphoreType.DMA,
            pltpu.SemaphoreType.DMA,
            pltpu.SemaphoreType.DMA,
            pltpu.SemaphoreType.DMA,
        ],
        compiler_params=pltpu.CompilerParams(needs_layout_passes=False,
                                             use_tc_tiling_on_sc=False),
    )


_NB3 = 400
_NG3 = N // _NB3


def _final_body(acc0_ref, acc1_ref, lo_ref, hi_ref, as_ref, ad_ref, b_ref,
                bias_ref, pw_ref, pb_ref, pooled_ref, s_ref,
                sums_ref, cnt_ref):
    i = pl.program_id(0)
    al = as_ref[...] + ad_ref[...]
    ex_self = jnp.exp(jnp.where(al >= 0.0, al, 0.2 * al))
    xh = jnp.concatenate([lo_ref[0], hi_ref[0]], axis=1)
    num = jnp.concatenate([acc0_ref[0, :, :DH], acc1_ref[0, :, :DH]], axis=1)
    num = num + ex_self * xh
    den = acc0_ref[0, :, DH:DH + 1] + ex_self + 1e-16
    xho = num / den + bias_ref[...]
    logits = jnp.sum(xho * pw_ref[...], axis=1, keepdims=True) + pb_ref[...]
    sc = 1.0 / (1.0 + jnp.exp(-logits))
    s_ref[...] = sc
    xw = sc * xho
    onehot = (b_ref[...] == lax.broadcasted_iota(jnp.int32, (_NB3, G), 1)
              ).astype(jnp.float32)
    psum = lax.dot_general(onehot, xw, (((0,), (0,)), ((), ())),
                           preferred_element_type=jnp.float32)
    pcnt = lax.dot_general(onehot, jnp.ones((_NB3, 1), jnp.float32),
                           (((0,), (0,)), ((), ())),
                           preferred_element_type=jnp.float32)

    @pl.when(i == 0)
    def _():
        sums_ref[...] = psum
        cnt_ref[...] = pcnt

    @pl.when(i > 0)
    def _():
        sums_ref[...] += psum
        cnt_ref[...] += pcnt

    @pl.when(i == _NG3 - 1)
    def _():
        pooled_ref[...] = sums_ref[...] / jnp.maximum(cnt_ref[...], 1.0)


def _finalize(acc, xh2, a_src, a_dst, batch2d, bias_row, pw_row, pb):
    return pl.pallas_call(
        _final_body,
        grid=(_NG3,),
        in_specs=[
            pl.BlockSpec((1, _NB3, ACCW), lambda i: (0, i, 0)),
            pl.BlockSpec((1, _NB3, ACCW), lambda i: (1, i, 0)),
            pl.BlockSpec((1, _NB3, DH), lambda i: (0, i, 0)),
            pl.BlockSpec((1, _NB3, DH), lambda i: (1, i, 0)),
            pl.BlockSpec((_NB3, 1), lambda i: (i, 0)),
            pl.BlockSpec((_NB3, 1), lambda i: (i, 0)),
            pl.BlockSpec((_NB3, 1), lambda i: (i, 0)),
            pl.BlockSpec((1, D), lambda i: (0, 0)),
            pl.BlockSpec((1, D), lambda i: (0, 0)),
            pl.BlockSpec((1, 1), lambda i: (0, 0)),
        ],
        out_specs=[
            pl.BlockSpec((G, D), lambda i: (0, 0)),
            pl.BlockSpec((_NB3, 1), lambda i: (i, 0)),
        ],
        out_shape=[
            jax.ShapeDtypeStruct((G, D), jnp.float32),
            jax.ShapeDtypeStruct((N, 1), jnp.float32),
        ],
        scratch_shapes=[
            pltpu.VMEM((G, D), jnp.float32),
            pltpu.VMEM((G, 1), jnp.float32),
        ],
    )(acc, acc, xh2, xh2, a_src, a_dst, batch2d, bias_row, pw_row, pb)


@jax.jit
def kernel(x, edge_index, batch, W, att_src, att_dst, bias, proj_W, proj_b):
    xh2, a_src, a_dst = _node_proj(
        x, W, att_src.reshape(1, D), att_dst.reshape(1, D))
    acc = _edge_pass()(edge_index, a_src.reshape(N), a_dst.reshape(N), xh2)
    pooled, scores = _finalize(
        acc, xh2, a_src, a_dst, batch.reshape(N, 1),
        bias.reshape(1, D), proj_W.reshape(1, D), proj_b.reshape(1, 1))
    return (pooled, scores)


# speedup vs baseline: 42.2838x; 1.7944x over previous
import functools

import jax
import jax.numpy as jnp
from jax import lax
from jax.experimental import pallas as pl
from jax.experimental.pallas import tpu as pltpu
from jax.experimental.pallas import tpu_sc as plsc

N = 10000
E = 320000
D = 128
DH = D // 2
G = 64
ACCW = 80
CH = 128
NCH = E // CH
NC = 2
NS = 16
NCHW = (NCH // NS) & ~1
NREM = NCH - NCHW * NS
NP = 10240
NR = NP // NS


_NB1 = 1000


def _node_proj_body(x_ref, w_ref, asv_ref, adv_ref, xh2_ref, s_ref, d_ref):
    xh = jnp.dot(x_ref[...], w_ref[...], preferred_element_type=jnp.float32)
    xh2_ref[0] = xh[:, :DH]
    xh2_ref[1] = xh[:, DH:]
    s_ref[...] = jnp.sum(xh * asv_ref[...], axis=1, keepdims=True)
    d_ref[...] = jnp.sum(xh * adv_ref[...], axis=1, keepdims=True)


def _node_proj(x, W, att_src_row, att_dst_row):
    return pl.pallas_call(
        _node_proj_body,
        grid=(N // _NB1,),
        in_specs=[
            pl.BlockSpec((_NB1, D), lambda i: (i, 0)),
            pl.BlockSpec((D, D), lambda i: (0, 0)),
            pl.BlockSpec((1, D), lambda i: (0, 0)),
            pl.BlockSpec((1, D), lambda i: (0, 0)),
        ],
        out_specs=[
            pl.BlockSpec((2, _NB1, DH), lambda i: (0, i, 0)),
            pl.BlockSpec((_NB1, 1), lambda i: (i, 0)),
            pl.BlockSpec((_NB1, 1), lambda i: (i, 0)),
        ],
        out_shape=[
            jax.ShapeDtypeStruct((2, N, DH), jnp.float32),
            jax.ShapeDtypeStruct((N, 1), jnp.float32),
            jax.ShapeDtypeStruct((N, 1), jnp.float32),
        ],
    )(x, W, att_src_row, att_dst_row)


def _edge_body(edge_ref, asrc_hbm, adst_hbm, xh2_hbm, out_ref,
               asrc_v, adst_v,
               src_b0, dst_b0, rows_b0, scaled_b0, dsc_b0,
               src_b1, dst_b1, rows_b1, scaled_b1, dsc_b1,
               ex_b, acc_sh, se0, se1, sg, ss0, ss1):
    c = lax.axis_index("c")
    s = lax.axis_index("s")

    pltpu.sync_copy(asrc_hbm, asrc_v)
    pltpu.sync_copy(adst_hbm, adst_v)

    def _zb(j, cy):
        for q in range(ACCW // 16):
            scaled_b0[j, pl.ds(16 * q, 16)] = jnp.zeros((16,), jnp.float32)
        return cy
    lax.fori_loop(0, CH, _zb, 0)
    r0 = s * NR
    for t in range(NR // CH):
        pltpu.sync_copy(scaled_b0, acc_sh.at[pl.ds(r0 + CH * t, CH)])
    plsc.subcore_barrier()

    bufs = ((src_b0, dst_b0, rows_b0, scaled_b0, dsc_b0, se0, ss0),
            (src_b1, dst_b1, rows_b1, scaled_b1, dsc_b1, se1, ss1))

    def _start_edges(i, sbuf, dbuf, se):
        off = (s + NS * i) * CH
        pltpu.async_copy(edge_ref.at[0, pl.ds(off, CH)], sbuf, se)
        pltpu.async_copy(edge_ref.at[1, pl.ds(off, CH)], dbuf, se)

    def _wait_edges(sbuf, dbuf, se):
        pltpu.make_async_copy(edge_ref.at[0, pl.ds(0, CH)], sbuf, se).wait()
        pltpu.make_async_copy(edge_ref.at[1, pl.ds(0, CH)], dbuf, se).wait()

    def _compute_ex(sbuf, dbuf):
        svs = [sbuf[pl.ds(16 * v, 16)] for v in range(CH // 16)]
        dvs = [dbuf[pl.ds(16 * v, 16)] for v in range(CH // 16)]
        avs = [plsc.load_gather(asrc_v, [sv]) for sv in svs]
        bvs = [plsc.load_gather(adst_v, [dv]) for dv in dvs]
        for v in range(CH // 16):
            al = avs[v] + bvs[v]
            al = jnp.where(al >= 0.0, al, 0.2 * al)
            ex_b[pl.ds(16 * v, 16)] = jnp.exp(al)

    RB = 4

    def _scale(rbuf, cbuf):
        lane = lax.iota(jnp.int32, 16)

        def _blk(jb, cy):
            j0 = RB * jb
            es = [plsc.load_gather(ex_b, [jnp.full((16,), j0 + t, jnp.int32)])
                  for t in range(RB)]
            vals = [[rbuf[j0 + t, pl.ds(16 * q, 16)]
                     for q in range(DH // 16)] for t in range(RB)]
            for t in range(RB):
                for q in range(DH // 16):
                    cbuf[j0 + t, pl.ds(16 * q, 16)] = es[t] * vals[t][q]
                cbuf[j0 + t, pl.ds(DH, 16)] = jnp.where(lane == 0, es[t], 0.0)
            return cy
        lax.fori_loop(0, CH // RB, _blk, 0, unroll=2)

    _start_edges(0, src_b0, dst_b0, se0)
    _start_edges(1, src_b1, dst_b1, se1)
    _wait_edges(src_b0, dst_b0, se0)
    pltpu.async_copy(xh2_hbm.at[c].at[src_b0], rows_b0, sg)

    def _pair(k, cy):
        for b in range(2):
            i = 2 * k + b
            sbuf, dbuf, rbuf, cbuf, dsc, se, ss = bufs[b]
            nsbuf, ndbuf, nrbuf, _, _, nse, _ = bufs[1 - b]

            @pl.when(k >= 1)
            def _():
                pltpu.make_async_copy(cbuf, acc_sh.at[dsc], ss).wait()
            pltpu.make_async_copy(xh2_hbm.at[c].at[sbuf], rbuf, sg).wait()
            _compute_ex(sbuf, dbuf)
            dvals = [dbuf[pl.ds(16 * v, 16)] for v in range(CH // 16)]
            for v in range(CH // 16):
                dsc[pl.ds(16 * v, 16)] = dvals[v]

            @pl.when(k < NCHW // 2 - 1)
            def _():
                _start_edges(i + 2, sbuf, dbuf, se)

            def _next_gather():
                _wait_edges(nsbuf, ndbuf, nse)
                pltpu.async_copy(xh2_hbm.at[c].at[nsbuf], nrbuf, sg)
            if b == 0:
                _next_gather()
            else:
                pl.when(k < NCHW // 2 - 1)(_next_gather)
            _scale(rbuf, cbuf)
            pltpu.async_copy(cbuf, acc_sh.at[dsc], ss, add=True)
        return cy
    lax.fori_loop(0, NCHW // 2, _pair, 0)

    pltpu.make_async_copy(scaled_b0, acc_sh.at[dsc_b0], ss0).wait()
    pltpu.make_async_copy(scaled_b1, acc_sh.at[dsc_b1], ss1).wait()

    @pl.when(s < NREM)
    def _():
        off = (NCHW * NS + s) * CH
        pltpu.sync_copy(edge_ref.at[0, pl.ds(off, CH)], src_b0)
        pltpu.sync_copy(edge_ref.at[1, pl.ds(off, CH)], dst_b0)
        pltpu.async_copy(xh2_hbm.at[c].at[src_b0], rows_b0, sg).wait()
        _compute_ex(src_b0, dst_b0)
        _scale(rows_b0, scaled_b0)
        pltpu.sync_copy(scaled_b0, acc_sh.at[dst_b0], add=True)

    plsc.subcore_barrier()
    for t in range(NR // CH):
        pltpu.sync_copy(acc_sh.at[pl.ds(r0 + CH * t, CH)], scaled_b0)
        pltpu.sync_copy(scaled_b0, out_ref.at[c, pl.ds(r0 + CH * t, CH)])


@functools.cache
def _edge_pass():
    return pl.kernel(
        _edge_body,
        mesh=plsc.VectorSubcoreMesh(core_axis_name="c", subcore_axis_name="s",
                                    num_cores=NC, num_subcores=NS),
        out_type=jax.ShapeDtypeStruct((NC, NP, ACCW), jnp.float32),
        scratch_types=[
            pltpu.VMEM((N,), jnp.float32),
            pltpu.VMEM((N,), jnp.float32),
            pltpu.VMEM((CH,), jnp.int32),
            pltpu.VMEM((CH,), jnp.int32),
            pltpu.VMEM((CH, DH), jnp.float32),
            pltpu.VMEM((CH, ACCW), jnp.float32),
            pltpu.VMEM((CH,), jnp.int32),
            pltpu.VMEM((CH,), jnp.int32),
            pltpu.VMEM((CH,), jnp.int32),
            pltpu.VMEM((CH, DH), jnp.float32),
            pltpu.VMEM((CH, ACCW), jnp.float32),
            pltpu.VMEM((CH,), jnp.int32),
            pltpu.VMEM((CH,), jnp.float32),
            pltpu.VMEM_SHARED((NP, ACCW), jnp.float32),
            pltpu.SemaphoreType.DMA,
            pltpu.SemaphoreType.DMA,
            pltpu.SemaphoreType.DMA,
            pltpu.SemaphoreType.DMA,
            pltpu.SemaphoreType.DMA,
        ],
        compiler_params=pltpu.CompilerParams(needs_layout_passes=False,
                                             use_tc_tiling_on_sc=False),
    )


_NB3 = 400
_NG3 = N // _NB3


def _final_body(acc0_ref, acc1_ref, lo_ref, hi_ref, as_ref, ad_ref, b_ref,
                bias_ref, pw_ref, pb_ref, pooled_ref, s_ref,
                sums_ref, cnt_ref):
    i = pl.program_id(0)
    al = as_ref[...] + ad_ref[...]
    ex_self = jnp.exp(jnp.where(al >= 0.0, al, 0.2 * al))
    xh = jnp.concatenate([lo_ref[0], hi_ref[0]], axis=1)
    num = jnp.concatenate([acc0_ref[0, :, :DH], acc1_ref[0, :, :DH]], axis=1)
    num = num + ex_self * xh
    den = acc0_ref[0, :, DH:DH + 1] + ex_self + 1e-16
    xho = num / den + bias_ref[...]
    logits = jnp.sum(xho * pw_ref[...], axis=1, keepdims=True) + pb_ref[...]
    sc = 1.0 / (1.0 + jnp.exp(-logits))
    s_ref[...] = sc
    xw = sc * xho
    onehot = (b_ref[...] == lax.broadcasted_iota(jnp.int32, (_NB3, G), 1)
              ).astype(jnp.float32)
    psum = lax.dot_general(onehot, xw, (((0,), (0,)), ((), ())),
                           preferred_element_type=jnp.float32)
    pcnt = lax.dot_general(onehot, jnp.ones((_NB3, 1), jnp.float32),
                           (((0,), (0,)), ((), ())),
                           preferred_element_type=jnp.float32)

    @pl.when(i == 0)
    def _():
        sums_ref[...] = psum
        cnt_ref[...] = pcnt

    @pl.when(i > 0)
    def _():
        sums_ref[...] += psum
        cnt_ref[...] += pcnt

    @pl.when(i == _NG3 - 1)
    def _():
        pooled_ref[...] = sums_ref[...] / jnp.maximum(cnt_ref[...], 1.0)


def _finalize(acc, xh2, a_src, a_dst, batch2d, bias_row, pw_row, pb):
    return pl.pallas_call(
        _final_body,
        grid=(_NG3,),
        in_specs=[
            pl.BlockSpec((1, _NB3, ACCW), lambda i: (0, i, 0)),
            pl.BlockSpec((1, _NB3, ACCW), lambda i: (1, i, 0)),
            pl.BlockSpec((1, _NB3, DH), lambda i: (0, i, 0)),
            pl.BlockSpec((1, _NB3, DH), lambda i: (1, i, 0)),
            pl.BlockSpec((_NB3, 1), lambda i: (i, 0)),
            pl.BlockSpec((_NB3, 1), lambda i: (i, 0)),
            pl.BlockSpec((_NB3, 1), lambda i: (i, 0)),
            pl.BlockSpec((1, D), lambda i: (0, 0)),
            pl.BlockSpec((1, D), lambda i: (0, 0)),
            pl.BlockSpec((1, 1), lambda i: (0, 0)),
        ],
        out_specs=[
            pl.BlockSpec((G, D), lambda i: (0, 0)),
            pl.BlockSpec((_NB3, 1), lambda i: (i, 0)),
        ],
        out_shape=[
            jax.ShapeDtypeStruct((G, D), jnp.float32),
            jax.ShapeDtypeStruct((N, 1), jnp.float32),
        ],
        scratch_shapes=[
            pltpu.VMEM((G, D), jnp.float32),
            pltpu.VMEM((G, 1), jnp.float32),
        ],
    )(acc, acc, xh2, xh2, a_src, a_dst, batch2d, bias_row, pw_row, pb)


@jax.jit
def kernel(x, edge_index, batch, W, att_src, att_dst, bias, proj_W, proj_b):
    xh2, a_src, a_dst = _node_proj(
        x, W, att_src.reshape(1, D), att_dst.reshape(1, D))
    acc = _edge_pass()(edge_index, a_src.reshape(N), a_dst.reshape(N), xh2)
    pooled, scores = _finalize(
        acc, xh2, a_src, a_dst, batch.reshape(N, 1),
        bias.reshape(1, D), proj_W.reshape(1, D), proj_b.reshape(1, 1))
    return (pooled, scores)
